# Initial kernel scaffold; baseline (speedup 1.0000x reference)
import functools

import jax
import jax.numpy as jnp
from jax import lax
from jax.experimental import pallas as pl
from jax.experimental.pallas import tpu as pltpu
from jax.experimental.pallas import tpu_sc as plsc

D = 32

NC = 2
NS = 16
NW = NC * NS

CHUNK = 128
GROUP = 8
GROUP_ROWS = CHUNK * GROUP


@functools.partial(jax.jit, static_argnums=0)
def _gather(B, xf, table):
    b_per_w = B // NW
    n_rows = b_per_w // CHUNK
    n_groups = n_rows // GROUP

    mesh = plsc.VectorSubcoreMesh(
        core_axis_name="c", subcore_axis_name="s",
        num_cores=NC, num_subcores=NS)

    @functools.partial(
        pl.kernel,
        out_type=jax.ShapeDtypeStruct((B, D), jnp.float32),
        mesh=mesh,
        scratch_types=[
            pltpu.VMEM((n_rows, CHUNK), jnp.int32),
            pltpu.VMEM((2, GROUP_ROWS, D), jnp.float32),
            pltpu.SemaphoreType.DMA,
            pltpu.SemaphoreType.DMA,
        ],
    )
    def k(x_hbm, table_hbm, out_hbm, idx_v, rows_v, gsem, wsem):
        wid = lax.axis_index("s") * NC + lax.axis_index("c")
        base = wid * b_per_w
        pltpu.sync_copy(x_hbm.at[wid], idx_v)

        def fire(g, buf):
            return [
                pltpu.async_copy(
                    table_hbm.at[idx_v.at[g * GROUP + b]],
                    rows_v.at[buf, pl.ds(b * CHUNK, CHUNK)],
                    gsem)
                for b in range(GROUP)
            ]

        def write(g, buf):
            return pltpu.async_copy(
                rows_v.at[buf],
                out_hbm.at[pl.ds(base + g * GROUP_ROWS, GROUP_ROWS)],
                wsem)

        pending = fire(0, 0)
        prev_write = None
        for g in range(n_groups):
            nxt = fire(g + 1, (g + 1) % 2) if g + 1 < n_groups else None
            for c in pending:
                c.wait()
            if prev_write is not None:
                prev_write.wait()
            prev_write = write(g, g % 2)
            pending = nxt
        prev_write.wait()

    return k(xf, table)


def kernel(x, table):
    B = x.size
    xf = x.reshape(-1).astype(jnp.int32).reshape(NW, B // (NW * CHUNK), CHUNK)
    out = _gather(B, xf, table)
    return out.reshape(*x.shape, D)


# speedup vs baseline: 1.1021x; 1.1021x over previous
import functools

import jax
import jax.numpy as jnp
from jax import lax
from jax.experimental import pallas as pl
from jax.experimental.pallas import tpu as pltpu
from jax.experimental.pallas import tpu_sc as plsc

D = 32

NC = 2
NS = 16
NW = NC * NS

CHUNK = 128
GROUP = 8
GROUP_ROWS = CHUNK * GROUP


@functools.partial(jax.jit, static_argnums=0)
def _gather(B, xf, table):
    b_per_w = B // NW
    n_rows = b_per_w // CHUNK
    n_groups = n_rows // GROUP

    mesh = plsc.VectorSubcoreMesh(
        core_axis_name="c", subcore_axis_name="s",
        num_cores=NC, num_subcores=NS)

    @functools.partial(
        pl.kernel,
        out_type=jax.ShapeDtypeStruct((B, D), jnp.float32),
        mesh=mesh,
        compiler_params=pltpu.CompilerParams(use_tc_tiling_on_sc=False),
        scratch_types=[
            pltpu.VMEM((n_rows, CHUNK), jnp.int32),
            pltpu.VMEM((GROUP_ROWS, D), jnp.float32),
            pltpu.SemaphoreType.DMA,
        ],
    )
    def k(x_hbm, table_hbm, out_hbm, idx_v, rows_v, gsem):
        wid = lax.axis_index("s") * NC + lax.axis_index("c")
        base = wid * b_per_w
        pltpu.sync_copy(x_hbm.at[wid], idx_v)

        @pl.loop(0, n_groups)
        def _(g):
            copies = [
                pltpu.async_copy(
                    table_hbm.at[idx_v.at[g * GROUP + b]],
                    rows_v.at[pl.ds(b * CHUNK, CHUNK)],
                    gsem)
                for b in range(GROUP)
            ]
            for c in copies:
                c.wait()
            pltpu.sync_copy(
                rows_v,
                out_hbm.at[pl.ds(base + g * GROUP_ROWS, GROUP_ROWS)])

    return k(xf, table)


def kernel(x, table):
    B = x.size
    xf = x.reshape(-1).astype(jnp.int32).reshape(NW, B // (NW * CHUNK), CHUNK)
    out = _gather(B, xf, table)
    return out.reshape(*x.shape, D)


# speedup vs baseline: 1.1124x; 1.0094x over previous
import functools

import jax
import jax.numpy as jnp
from jax import lax
from jax.experimental import pallas as pl
from jax.experimental.pallas import tpu as pltpu
from jax.experimental.pallas import tpu_sc as plsc

D = 32

NC = 2
NS = 16
NW = NC * NS

CHUNK = 128
GROUP = 5
GROUP_ROWS = CHUNK * GROUP
NBUF = 3


@functools.partial(jax.jit, static_argnums=0)
def _gather(B, xf, table):
    b_per_w = B // NW
    n_rows = b_per_w // CHUNK
    n_groups = n_rows // GROUP
    assert n_groups >= 2 * NBUF and (n_groups - 4) % NBUF == 0

    mesh = plsc.VectorSubcoreMesh(
        core_axis_name="c", subcore_axis_name="s",
        num_cores=NC, num_subcores=NS)

    @functools.partial(
        pl.kernel,
        out_type=jax.ShapeDtypeStruct((B, D), jnp.float32),
        mesh=mesh,
        compiler_params=pltpu.CompilerParams(use_tc_tiling_on_sc=False),
        scratch_types=[
            pltpu.VMEM((n_rows, CHUNK), jnp.int32),
            pltpu.VMEM((NBUF, GROUP_ROWS, D), jnp.float32),
            [pltpu.SemaphoreType.DMA] * NBUF,
            [pltpu.SemaphoreType.DMA] * NBUF,
        ],
    )
    def k(x_hbm, table_hbm, out_hbm, idx_v, rows_v, gsems, wsems):
        wid = lax.axis_index("s") * NC + lax.axis_index("c")
        base = wid * b_per_w
        pltpu.sync_copy(x_hbm.at[wid], idx_v)

        def fire(g, buf):
            for b in range(GROUP):
                pltpu.async_copy(
                    table_hbm.at[idx_v.at[g * GROUP + b]],
                    rows_v.at[buf, pl.ds(b * CHUNK, CHUNK)],
                    gsems[buf])

        def drain(buf):
            pltpu.make_async_copy(
                table_hbm.at[pl.ds(0, GROUP_ROWS)],
                rows_v.at[buf],
                gsems[buf]).wait()

        def write(g, buf):
            pltpu.async_copy(
                rows_v.at[buf],
                out_hbm.at[pl.ds(base + g * GROUP_ROWS, GROUP_ROWS)],
                wsems[buf])

        def wait_write(buf):
            pltpu.make_async_copy(
                rows_v.at[buf],
                out_hbm.at[pl.ds(base, GROUP_ROWS)],
                wsems[buf]).wait()

        fire(0, 0)
        fire(1, 1)
        drain(0)
        fire(2, 2)
        write(0, 0)
        drain(1)
        wait_write(0)
        fire(3, 0)
        write(1, 1)

        @pl.loop(2, n_groups - 2, step=NBUF)
        def _(g0):
            for j in range(NBUF):
                g = g0 + j
                buf = (2 + j) % NBUF
                drain(buf)
                wait_write((buf + NBUF - 1) % NBUF)
                fire(g + 2, (buf + NBUF - 1) % NBUF)
                write(g, buf)

        gl = n_groups - 2
        drain(gl % NBUF)
        wait_write((gl - 1) % NBUF)
        write(gl, gl % NBUF)
        gl = n_groups - 1
        drain(gl % NBUF)
        wait_write((gl - 1) % NBUF)
        write(gl, gl % NBUF)
        wait_write(gl % NBUF)

    return k(xf, table)


def kernel(x, table):
    B = x.size
    xf = x.reshape(-1).astype(jnp.int32).reshape(NW, B // (NW * CHUNK), CHUNK)
    out = _gather(B, xf, table)
    return out.reshape(*x.shape, D)
